# baseline (speedup 1.0000x reference)
import functools

import jax
import jax.numpy as jnp
from jax.experimental import pallas as pl

B, R, L, IN, OUT = 128, 6, 128, 128, 64
NB = 4


def _aergcn_body(text_ref, adj_ref, w2_ref, sw_ref, sb_ref, out_ref):
    x = text_ref[...].reshape(NB * L, IN)
    w2 = w2_ref[...]
    sw = sw_ref[...]
    sb = sb_ref[0, 0]
    hid = jnp.dot(x, w2, preferred_element_type=jnp.float32)
    for b in range(NB):
        h_b = hid[b * L:(b + 1) * L]
        acc = jnp.zeros((L, OUT), jnp.float32)
        zsum = jnp.zeros((L, 1), jnp.float32)
        mrun = jnp.full((L, 1), -1e30, jnp.float32)
        for r in range(R):
            a = adj_ref[b, r]
            msg = jnp.dot(a, h_b[:, r * OUT:(r + 1) * OUT],
                          preferred_element_type=jnp.float32)
            den = jnp.sum(a, axis=1, keepdims=True)
            den = jnp.where(den == 0.0, 1.0, den)
            div = msg / den
            s = jnp.sum(div * sw, axis=1, keepdims=True) + sb
            mnew = jnp.maximum(mrun, s)
            corr = jnp.exp(mrun - mnew)
            e = jnp.exp(s - mnew)
            zsum = zsum * corr + e
            acc = acc * corr + e * div
            mrun = mnew
        out_ref[b] = acc / zsum


@jax.jit
def kernel(text, adj, weight, score_w, score_b):
    w2 = weight.transpose(1, 0, 2).reshape(IN, R * OUT)
    sb = score_b.reshape(1, 1)
    grid = (B // NB,)
    return pl.pallas_call(
        _aergcn_body,
        grid=grid,
        in_specs=[
            pl.BlockSpec((NB, L, IN), lambda i: (i, 0, 0)),
            pl.BlockSpec((NB, R, L, L), lambda i: (i, 0, 0, 0)),
            pl.BlockSpec((IN, R * OUT), lambda i: (0, 0)),
            pl.BlockSpec((1, OUT), lambda i: (0, 0)),
            pl.BlockSpec((1, 1), lambda i: (0, 0)),
        ],
        out_specs=pl.BlockSpec((NB, L, OUT), lambda i: (i, 0, 0)),
        out_shape=jax.ShapeDtypeStruct((B, L, OUT), jnp.float32),
    )(text, adj, w2, score_w, sb)
